# baseline (speedup 1.0000x reference)
import functools

import jax
import jax.numpy as jnp
from jax import lax
from jax.experimental import pallas as pl
from jax.experimental.pallas import tpu as pltpu
from jax.experimental.pallas import tpu_sc as plsc

N = 50000
E = 800000
F = 64
G = 64

NB = 64
RB = 784
N2 = NB * RB
ET = 2048
EPAD = 802816


def _layer1_body(gs_ref, gd_ref, w1a_ref, w1b_ref, b1_ref, w2_ref, b2_ref,
                 wdb_ref, m_ref, db_ref):
    gs = gs_ref[...]
    delta = gs - gd_ref[...]
    m1 = jax.lax.dot(gs, w1a_ref[...], preferred_element_type=jnp.float32)
    m1 = m1 + jax.lax.dot(delta, w1b_ref[...],
                          preferred_element_type=jnp.float32)
    m1 = jnp.maximum(m1 + b1_ref[...], 0.0)
    m_ref[...] = jax.lax.dot(m1, w2_ref[...],
                             preferred_element_type=jnp.float32) + b2_ref[...]
    db_ref[...] = jax.lax.dot(delta, wdb_ref[...],
                              preferred_element_type=jnp.float32)


def _layer1_mlp(gs, gd, w1a, w1b, b1, w2, b2, wdb):
    return pl.pallas_call(
        _layer1_body,
        grid=(EPAD // ET,),
        in_specs=[
            pl.BlockSpec((ET, 128), lambda i: (i, 0)),
            pl.BlockSpec((ET, 128), lambda i: (i, 0)),
            pl.BlockSpec((128, F), lambda i: (0, 0)),
            pl.BlockSpec((128, F), lambda i: (0, 0)),
            pl.BlockSpec((1, F), lambda i: (0, 0)),
            pl.BlockSpec((F, F), lambda i: (0, 0)),
            pl.BlockSpec((1, F), lambda i: (0, 0)),
            pl.BlockSpec((128, F), lambda i: (0, 0)),
        ],
        out_specs=[
            pl.BlockSpec((ET, F), lambda i: (i, 0)),
            pl.BlockSpec((ET, F), lambda i: (i, 0)),
        ],
        out_shape=[
            jax.ShapeDtypeStruct((EPAD, F), jnp.float32),
            jax.ShapeDtypeStruct((EPAD, F), jnp.float32),
        ],
    )(gs, gd, w1a, w1b, b1, w2, b2, wdb)


def _edge_mlp_body(xsrc_ref, db_ref, w1a_ref, b1_ref, w2_ref, b2_ref, out_ref):
    m1 = jax.lax.dot(xsrc_ref[...], w1a_ref[...],
                     preferred_element_type=jnp.float32)
    m1 = jnp.maximum(m1 + db_ref[...] + b1_ref[...], 0.0)
    out_ref[...] = jax.lax.dot(m1, w2_ref[...],
                               preferred_element_type=jnp.float32) + b2_ref[...]


def _edge_mlp(xsrc, db, w1a, b1, w2, b2):
    return pl.pallas_call(
        _edge_mlp_body,
        grid=(EPAD // ET,),
        in_specs=[
            pl.BlockSpec((ET, 128), lambda i: (i, 0)),
            pl.BlockSpec((ET, F), lambda i: (i, 0)),
            pl.BlockSpec((128, F), lambda i: (0, 0)),
            pl.BlockSpec((1, F), lambda i: (0, 0)),
            pl.BlockSpec((F, F), lambda i: (0, 0)),
            pl.BlockSpec((1, F), lambda i: (0, 0)),
        ],
        out_specs=pl.BlockSpec((ET, F), lambda i: (i, 0)),
        out_shape=jax.ShapeDtypeStruct((EPAD, F), jnp.float32),
    )(xsrc, db, w1a, b1, w2, b2)


def _any16(v):
    pc = plsc.all_reduce_population_count(v)
    return lax.squeeze(lax.slice(pc, (0,), (1,)), (0,))


_mesh = plsc.VectorSubcoreMesh(core_axis_name="c", subcore_axis_name="s")


@functools.partial(
    pl.kernel,
    mesh=_mesh,
    compiler_params=pltpu.CompilerParams(needs_layout_passes=False),
    out_type=jax.ShapeDtypeStruct((N2 * F,), jnp.float32),
    scratch_types=[
        pltpu.VMEM((RB * F,), jnp.float32),
        pltpu.VMEM((16, 16, 128), jnp.float32),
        pltpu.VMEM((16, 128), jnp.int32),
        pltpu.VMEM((NB * 16,), jnp.int32),
        pltpu.VMEM((NB * 16,), jnp.int32),
        pltpu.VMEM((RB,), jnp.int32),
        pltpu.VMEM((16,), jnp.int32),
        pltpu.SemaphoreType.DMA,
        pltpu.SemaphoreType.DMA,
        pltpu.SemaphoreType.DMA,
    ],
)
def _segmax(m3, d3, blo, bhi, zflat, out, acc, mbuf, dbuf, blov, bhiv, tmp,
            remref, sem, sem2, sem3):
    c_ax = lax.axis_index("c")
    s_ax = lax.axis_index("s")
    wid = s_ax * 2 + c_ax
    iota = lax.iota(jnp.int32, 16)

    pltpu.sync_copy(blo, blov)
    pltpu.sync_copy(bhi, bhiv)

    for ib in range(2):
        b = wid * 2 + ib
        e_lo = blov[pl.ds(b * 16, 16)]
        e_hi = bhiv[pl.ds(b * 16, 16)]
        pltpu.async_copy(zflat, acc, sem3).wait()

        sbv = lax.shift_right_logical(e_lo, 11)
        row0d = sbv * 16
        row0m = sbv * 64

        def sb_body(carry, e_lo=e_lo, e_hi=e_hi, row0d=row0d, row0m=row0m):
            k, flag = carry
            pltpu.async_copy(d3.at[row0d + k * 16 + iota], dbuf, sem2).wait()
            base_e = (row0d + k * 16) * 128

            for c in range(4):
                pltpu.async_copy(m3.at[row0m + k * 64 + c * 16 + iota],
                                 mbuf, sem).wait()

                def g_body(g2, _, c=c, k=k, base_e=base_e, e_lo=e_lo,
                           e_hi=e_hi):
                    gg = c * 32 + g2
                    dv = dbuf[gg // 8, pl.ds((gg % 8) * 16, 16)]
                    e5 = g2 * 16 + iota
                    ev = base_e + c * 512 + e5
                    valid = (ev >= e_lo) & (ev < e_hi)
                    r_idx = lax.shift_right_logical(e5, 5)
                    s_idx = lax.shift_right_logical(e5 & 31, 1)
                    l0 = (e5 & 1) * 64
                    dvb = dv * 64
                    remref[...] = valid.astype(jnp.int32)

                    def wcond2(flag2):
                        return flag2 != 0

                    def wbody2(flag2):
                        rem = remref[...] != 0
                        plsc.store_scatter(tmp, [dv], iota, mask=rem)
                        t = plsc.load_gather(tmp, [dv], mask=rem)
                        win = rem & (t == iota)
                        for j in range(F):
                            mv = plsc.load_gather(
                                mbuf, [r_idx, s_idx, l0 + j], mask=win)
                            av = plsc.load_gather(acc, [dvb + j], mask=win)
                            plsc.store_scatter(acc, [dvb + j],
                                               jnp.maximum(av, mv), mask=win)
                        nrem = rem & jnp.logical_not(win)
                        remref[...] = nrem.astype(jnp.int32)
                        return _any16(nrem)

                    lax.while_loop(wcond2, wbody2, _any16(valid))
                    return 0

                lax.fori_loop(0, 32, g_body, 0)

            nflag = _any16(base_e + 2048 < e_hi)
            return (k + 1, nflag)

        flag0 = _any16(row0d * 128 < e_hi)
        lax.while_loop(lambda cr: cr[1] != 0, sb_body, (jnp.int32(0), flag0))

        pltpu.sync_copy(acc, out.at[pl.ds(b * (RB * F), RB * F)])


_BW = EPAD // 32
_CH = 512


@functools.partial(
    pl.kernel,
    mesh=_mesh,
    compiler_params=pltpu.CompilerParams(needs_layout_passes=False),
    out_type=jax.ShapeDtypeStruct((EPAD, 128), jnp.float32),
    scratch_types=[
        pltpu.VMEM((_CH,), jnp.int32),
        pltpu.VMEM((_CH, 128), jnp.float32),
        pltpu.SemaphoreType.DMA,
    ],
)
def _gather128(table, idx, out, idxv, rows, sem):
    c_ax = lax.axis_index("c")
    s_ax = lax.axis_index("s")
    wid = s_ax * 2 + c_ax
    base = wid * _BW

    def body(k, _):
        off = base + k * _CH
        pltpu.sync_copy(idx.at[pl.ds(off, _CH)], idxv)
        pltpu.async_copy(table.at[idxv], rows, sem).wait()
        pltpu.sync_copy(rows, out.at[pl.ds(off, _CH)])
        return 0

    lax.fori_loop(0, _BW // _CH, body, 0)


def kernel(pos, edge_index, batch, c1_w1, c1_b1, c1_w2, c1_b2, c2_w1, c2_b1,
           c2_w2, c2_b2, r1_w, r1_b, r2_w, r2_b):
    mask = jnp.zeros((N,), dtype=bool).at[edge_index.reshape(-1)].set(True)
    assoc = jnp.cumsum(mask.astype(jnp.int32)) - 1
    ei = assoc[edge_index]
    src, dst = ei[0], ei[1]

    bucket = dst // RB
    perm = jnp.argsort(bucket, stable=True)
    pad_idx = (jnp.arange(EPAD - E, dtype=jnp.int32) % N2)
    src_p = jnp.concatenate([src[perm], pad_idx])
    dst_p = jnp.concatenate([dst[perm], pad_idx])
    dstl_p = jnp.concatenate([(dst - bucket * RB)[perm],
                              jnp.zeros((EPAD - E,), jnp.int32)])
    sb = bucket[perm]
    bstart = jnp.searchsorted(sb, jnp.arange(NB + 1, dtype=jnp.int32)
                              ).astype(jnp.int32)
    blo = jnp.repeat(bstart[:NB], 16)
    bhi = jnp.repeat(bstart[1:], 16)
    d3 = dstl_p.reshape(EPAD // 128, 128)
    zflat = jnp.zeros((RB * F,), jnp.float32)

    def segmax(m):
        hflat = _segmax(m.reshape(EPAD // 32, 16, 128), d3, blo, bhi, zflat)
        return hflat.reshape(N2, F)

    pos128 = jnp.zeros((N2, 128), jnp.float32).at[:N, :3].set(pos)
    gsrc = _gather128(pos128, src_p)
    gdst = _gather128(pos128, dst_p)
    w1a_1 = jnp.zeros((128, F), jnp.float32).at[:3].set(c1_w1[:3])
    w1b_1 = jnp.zeros((128, F), jnp.float32).at[:3].set(c1_w1[3:])
    wdb2 = jnp.zeros((128, F), jnp.float32).at[:3].set(c2_w1[F:])
    m1, db2 = _layer1_mlp(gsrc, gdst, w1a_1, w1b_1, c1_b1[None, :], c1_w2,
                          c1_b2[None, :], wdb2)
    h = segmax(m1)

    w1a_2 = jnp.zeros((128, F), jnp.float32).at[:F].set(c2_w1[:F])
    for _ in range(2):
        h128 = jnp.zeros((N2, 128), jnp.float32).at[:, :F].set(h)
        xsrc = _gather128(h128, src_p)
        m = _edge_mlp(xsrc, db2, w1a_2, c2_b1[None, :], c2_w2, c2_b2[None, :])
        h = segmax(m)

    hf = h[:N]
    sums = jax.ops.segment_sum(hf, batch, num_segments=G)
    cnt = jax.ops.segment_sum(jnp.ones((N,), jnp.float32), batch,
                              num_segments=G)
    mean = sums / jnp.maximum(cnt, 1.0)[:, None]
    return (mean @ r1_w + r1_b, mean @ r2_w + r2_b)


# speedup vs baseline: 1.0788x; 1.0788x over previous
import functools

import jax
import jax.numpy as jnp
from jax import lax
from jax.experimental import pallas as pl
from jax.experimental.pallas import tpu as pltpu
from jax.experimental.pallas import tpu_sc as plsc

N = 50000
E = 800000
F = 64
G = 64

NB = 64
RB = 784
N2 = NB * RB
ET = 2048
EPAD = 802816


def _layer1_body(gs_ref, gd_ref, w1a_ref, w1b_ref, b1_ref, w2_ref, b2_ref,
                 wdb_ref, m_ref, db_ref):
    gs = gs_ref[...]
    delta = gs - gd_ref[...]
    m1 = jax.lax.dot(gs, w1a_ref[...], preferred_element_type=jnp.float32)
    m1 = m1 + jax.lax.dot(delta, w1b_ref[...],
                          preferred_element_type=jnp.float32)
    m1 = jnp.maximum(m1 + b1_ref[...], 0.0)
    m_ref[...] = jax.lax.dot(m1, w2_ref[...],
                             preferred_element_type=jnp.float32) + b2_ref[...]
    db_ref[...] = jax.lax.dot(delta, wdb_ref[...],
                              preferred_element_type=jnp.float32)


def _layer1_mlp(gs, gd, w1a, w1b, b1, w2, b2, wdb):
    return pl.pallas_call(
        _layer1_body,
        grid=(EPAD // ET,),
        in_specs=[
            pl.BlockSpec((ET, 128), lambda i: (i, 0)),
            pl.BlockSpec((ET, 128), lambda i: (i, 0)),
            pl.BlockSpec((128, F), lambda i: (0, 0)),
            pl.BlockSpec((128, F), lambda i: (0, 0)),
            pl.BlockSpec((1, F), lambda i: (0, 0)),
            pl.BlockSpec((F, F), lambda i: (0, 0)),
            pl.BlockSpec((1, F), lambda i: (0, 0)),
            pl.BlockSpec((128, F), lambda i: (0, 0)),
        ],
        out_specs=[
            pl.BlockSpec((ET, F), lambda i: (i, 0)),
            pl.BlockSpec((ET, F), lambda i: (i, 0)),
        ],
        out_shape=[
            jax.ShapeDtypeStruct((EPAD, F), jnp.float32),
            jax.ShapeDtypeStruct((EPAD, F), jnp.float32),
        ],
    )(gs, gd, w1a, w1b, b1, w2, b2, wdb)


def _edge_mlp_body(xsrc_ref, db_ref, w1a_ref, b1_ref, w2_ref, b2_ref, out_ref):
    m1 = jax.lax.dot(xsrc_ref[...], w1a_ref[...],
                     preferred_element_type=jnp.float32)
    m1 = jnp.maximum(m1 + db_ref[...] + b1_ref[...], 0.0)
    out_ref[...] = jax.lax.dot(m1, w2_ref[...],
                               preferred_element_type=jnp.float32) + b2_ref[...]


def _edge_mlp(xsrc, db, w1a, b1, w2, b2):
    return pl.pallas_call(
        _edge_mlp_body,
        grid=(EPAD // ET,),
        in_specs=[
            pl.BlockSpec((ET, 128), lambda i: (i, 0)),
            pl.BlockSpec((ET, F), lambda i: (i, 0)),
            pl.BlockSpec((128, F), lambda i: (0, 0)),
            pl.BlockSpec((1, F), lambda i: (0, 0)),
            pl.BlockSpec((F, F), lambda i: (0, 0)),
            pl.BlockSpec((1, F), lambda i: (0, 0)),
        ],
        out_specs=pl.BlockSpec((ET, F), lambda i: (i, 0)),
        out_shape=jax.ShapeDtypeStruct((EPAD, F), jnp.float32),
    )(xsrc, db, w1a, b1, w2, b2)


def _any16(v):
    pc = plsc.all_reduce_population_count(v)
    return lax.squeeze(lax.slice(pc, (0,), (1,)), (0,))


_mesh = plsc.VectorSubcoreMesh(core_axis_name="c", subcore_axis_name="s")


@functools.partial(
    pl.kernel,
    mesh=_mesh,
    compiler_params=pltpu.CompilerParams(needs_layout_passes=False),
    out_type=jax.ShapeDtypeStruct((N2 * F,), jnp.float32),
    scratch_types=[
        pltpu.VMEM((RB * F,), jnp.float32),
        pltpu.VMEM((16, 16, 128), jnp.float32),
        pltpu.VMEM((16, 128), jnp.int32),
        pltpu.VMEM((NB * 16,), jnp.int32),
        pltpu.VMEM((NB * 16,), jnp.int32),
        pltpu.VMEM((RB,), jnp.int32),
        pltpu.VMEM((16,), jnp.int32),
        pltpu.SemaphoreType.DMA,
        pltpu.SemaphoreType.DMA,
        pltpu.SemaphoreType.DMA,
    ],
)
def _segmax(m3, d3, blo, bhi, zflat, out, acc, mbuf, dbuf, blov, bhiv, tmp,
            remref, sem, sem2, sem3):
    c_ax = lax.axis_index("c")
    s_ax = lax.axis_index("s")
    wid = s_ax * 2 + c_ax
    iota = lax.iota(jnp.int32, 16)

    pltpu.sync_copy(blo, blov)
    pltpu.sync_copy(bhi, bhiv)

    for ib in range(2):
        b = wid * 2 + ib
        e_lo = blov[pl.ds(b * 16, 16)]
        e_hi = bhiv[pl.ds(b * 16, 16)]
        pltpu.async_copy(zflat, acc, sem3).wait()

        sbv = lax.shift_right_logical(e_lo, 11)
        row0d = sbv * 16
        row0m = sbv * 64

        def sb_body(carry, e_lo=e_lo, e_hi=e_hi, row0d=row0d, row0m=row0m):
            k, flag = carry
            pltpu.async_copy(d3.at[row0d + k * 16 + iota], dbuf, sem2).wait()
            base_e = (row0d + k * 16) * 128

            for c in range(4):
                pltpu.async_copy(m3.at[row0m + k * 64 + c * 16 + iota],
                                 mbuf, sem).wait()

                def g_body(g2, _, c=c, k=k, base_e=base_e, e_lo=e_lo,
                           e_hi=e_hi):
                    gg = c * 32 + g2
                    dv = dbuf[gg // 8, pl.ds((gg % 8) * 16, 16)]
                    e5 = g2 * 16 + iota
                    ev = base_e + c * 512 + e5
                    valid = (ev >= e_lo) & (ev < e_hi)
                    r_idx = lax.shift_right_logical(e5, 5)
                    s_idx = lax.shift_right_logical(e5 & 31, 1)
                    l0 = (e5 & 1) * 64
                    dvb = dv * 64
                    remref[...] = valid.astype(jnp.int32)

                    def wcond2(flag2):
                        return flag2 != 0

                    def wbody2(flag2):
                        rem = remref[...] != 0
                        plsc.store_scatter(tmp, [dv], iota, mask=rem)
                        t = plsc.load_gather(tmp, [dv], mask=rem)
                        win = rem & (t == iota)
                        for j0 in range(0, F, 8):
                            mvs = [plsc.load_gather(
                                mbuf, [r_idx, s_idx, l0 + j], mask=win)
                                for j in range(j0, j0 + 8)]
                            avs = [plsc.load_gather(acc, [dvb + j], mask=win)
                                   for j in range(j0, j0 + 8)]
                            for dj in range(8):
                                plsc.store_scatter(
                                    acc, [dvb + (j0 + dj)],
                                    jnp.maximum(avs[dj], mvs[dj]), mask=win)
                        nrem = rem & jnp.logical_not(win)
                        remref[...] = nrem.astype(jnp.int32)
                        return _any16(nrem)

                    lax.while_loop(wcond2, wbody2, _any16(valid))
                    return 0

                lax.fori_loop(0, 32, g_body, 0)

            nflag = _any16(base_e + 2048 < e_hi)
            return (k + 1, nflag)

        flag0 = _any16(row0d * 128 < e_hi)
        lax.while_loop(lambda cr: cr[1] != 0, sb_body, (jnp.int32(0), flag0))

        pltpu.sync_copy(acc, out.at[pl.ds(b * (RB * F), RB * F)])


_BW = EPAD // 32
_CH = 512


@functools.partial(
    pl.kernel,
    mesh=_mesh,
    compiler_params=pltpu.CompilerParams(needs_layout_passes=False),
    out_type=jax.ShapeDtypeStruct((EPAD, 128), jnp.float32),
    scratch_types=[
        pltpu.VMEM((_CH,), jnp.int32),
        pltpu.VMEM((_CH, 128), jnp.float32),
        pltpu.SemaphoreType.DMA,
    ],
)
def _gather128(table, idx, out, idxv, rows, sem):
    c_ax = lax.axis_index("c")
    s_ax = lax.axis_index("s")
    wid = s_ax * 2 + c_ax
    base = wid * _BW

    def body(k, _):
        off = base + k * _CH
        pltpu.sync_copy(idx.at[pl.ds(off, _CH)], idxv)
        pltpu.async_copy(table.at[idxv], rows, sem).wait()
        pltpu.sync_copy(rows, out.at[pl.ds(off, _CH)])
        return 0

    lax.fori_loop(0, _BW // _CH, body, 0)


def kernel(pos, edge_index, batch, c1_w1, c1_b1, c1_w2, c1_b2, c2_w1, c2_b1,
           c2_w2, c2_b2, r1_w, r1_b, r2_w, r2_b):
    mask = jnp.zeros((N,), dtype=bool).at[edge_index.reshape(-1)].set(True)
    assoc = jnp.cumsum(mask.astype(jnp.int32)) - 1
    ei = assoc[edge_index]
    src, dst = ei[0], ei[1]

    bucket = dst // RB
    skey = jnp.sort(bucket * 1048576 +
                    jnp.arange(E, dtype=jnp.int32))
    perm = skey & 0xFFFFF
    pad_idx = (jnp.arange(EPAD - E, dtype=jnp.int32) % N2)
    src_p = jnp.concatenate([src[perm], pad_idx])
    dst_p = jnp.concatenate([dst[perm], pad_idx])
    dstl_p = jnp.concatenate([(dst - bucket * RB)[perm],
                              jnp.zeros((EPAD - E,), jnp.int32)])
    sb = skey >> 20
    bstart = jnp.searchsorted(sb, jnp.arange(NB + 1, dtype=jnp.int32)
                              ).astype(jnp.int32)
    blo = jnp.repeat(bstart[:NB], 16)
    bhi = jnp.repeat(bstart[1:], 16)
    d3 = dstl_p.reshape(EPAD // 128, 128)
    zflat = jnp.zeros((RB * F,), jnp.float32)

    def segmax(m):
        hflat = _segmax(m.reshape(EPAD // 32, 16, 128), d3, blo, bhi, zflat)
        return hflat.reshape(N2, F)

    pos128 = jnp.zeros((N2, 128), jnp.float32).at[:N, :3].set(pos)
    gsrc = _gather128(pos128, src_p)
    gdst = _gather128(pos128, dst_p)
    w1a_1 = jnp.zeros((128, F), jnp.float32).at[:3].set(c1_w1[:3])
    w1b_1 = jnp.zeros((128, F), jnp.float32).at[:3].set(c1_w1[3:])
    wdb2 = jnp.zeros((128, F), jnp.float32).at[:3].set(c2_w1[F:])
    m1, db2 = _layer1_mlp(gsrc, gdst, w1a_1, w1b_1, c1_b1[None, :], c1_w2,
                          c1_b2[None, :], wdb2)
    h = segmax(m1)

    w1a_2 = jnp.zeros((128, F), jnp.float32).at[:F].set(c2_w1[:F])
    for _ in range(2):
        h128 = jnp.zeros((N2, 128), jnp.float32).at[:, :F].set(h)
        xsrc = _gather128(h128, src_p)
        m = _edge_mlp(xsrc, db2, w1a_2, c2_b1[None, :], c2_w2, c2_b2[None, :])
        h = segmax(m)

    hf = h[:N]
    sums = jax.ops.segment_sum(hf, batch, num_segments=G)
    cnt = jax.ops.segment_sum(jnp.ones((N,), jnp.float32), batch,
                              num_segments=G)
    mean = sums / jnp.maximum(cnt, 1.0)[:, None]
    return (mean @ r1_w + r1_b, mean @ r2_w + r2_b)


# speedup vs baseline: 1.5727x; 1.4577x over previous
import functools

import jax
import jax.numpy as jnp
from jax import lax
from jax.experimental import pallas as pl
from jax.experimental.pallas import tpu as pltpu
from jax.experimental.pallas import tpu_sc as plsc

N = 50000
E = 800000
F = 64
G = 64

NB = 64
RB = 784
N2 = NB * RB
ET = 2048
EPAD = 802816


def _layer1_body(gs_ref, gd_ref, w1a_ref, w1b_ref, b1_ref, w2_ref, b2_ref,
                 wdb_ref, m_ref, db_ref):
    gs = gs_ref[...]
    delta = gs - gd_ref[...]
    m1 = jax.lax.dot(gs, w1a_ref[...], preferred_element_type=jnp.float32)
    m1 = m1 + jax.lax.dot(delta, w1b_ref[...],
                          preferred_element_type=jnp.float32)
    m1 = jnp.maximum(m1 + b1_ref[...], 0.0)
    m_ref[...] = jax.lax.dot(m1, w2_ref[...],
                             preferred_element_type=jnp.float32) + b2_ref[...]
    db_ref[...] = jax.lax.dot(delta, wdb_ref[...],
                              preferred_element_type=jnp.float32)


def _layer1_mlp(gs, gd, w1a, w1b, b1, w2, b2, wdb):
    return pl.pallas_call(
        _layer1_body,
        grid=(EPAD // ET,),
        in_specs=[
            pl.BlockSpec((ET, 128), lambda i: (i, 0)),
            pl.BlockSpec((ET, 128), lambda i: (i, 0)),
            pl.BlockSpec((128, F), lambda i: (0, 0)),
            pl.BlockSpec((128, F), lambda i: (0, 0)),
            pl.BlockSpec((1, F), lambda i: (0, 0)),
            pl.BlockSpec((F, F), lambda i: (0, 0)),
            pl.BlockSpec((1, F), lambda i: (0, 0)),
            pl.BlockSpec((128, F), lambda i: (0, 0)),
        ],
        out_specs=[
            pl.BlockSpec((ET, F), lambda i: (i, 0)),
            pl.BlockSpec((ET, F), lambda i: (i, 0)),
        ],
        out_shape=[
            jax.ShapeDtypeStruct((EPAD, F), jnp.float32),
            jax.ShapeDtypeStruct((EPAD, F), jnp.float32),
        ],
    )(gs, gd, w1a, w1b, b1, w2, b2, wdb)


def _edge_mlp_body(xsrc_ref, db_ref, w1a_ref, b1_ref, w2_ref, b2_ref, out_ref):
    m1 = jax.lax.dot(xsrc_ref[...], w1a_ref[...],
                     preferred_element_type=jnp.float32)
    m1 = jnp.maximum(m1 + db_ref[...] + b1_ref[...], 0.0)
    out_ref[...] = jax.lax.dot(m1, w2_ref[...],
                               preferred_element_type=jnp.float32) + b2_ref[...]


def _edge_mlp(xsrc, db, w1a, b1, w2, b2):
    return pl.pallas_call(
        _edge_mlp_body,
        grid=(EPAD // ET,),
        in_specs=[
            pl.BlockSpec((ET, 128), lambda i: (i, 0)),
            pl.BlockSpec((ET, F), lambda i: (i, 0)),
            pl.BlockSpec((128, F), lambda i: (0, 0)),
            pl.BlockSpec((1, F), lambda i: (0, 0)),
            pl.BlockSpec((F, F), lambda i: (0, 0)),
            pl.BlockSpec((1, F), lambda i: (0, 0)),
        ],
        out_specs=pl.BlockSpec((ET, F), lambda i: (i, 0)),
        out_shape=jax.ShapeDtypeStruct((EPAD, F), jnp.float32),
    )(xsrc, db, w1a, b1, w2, b2)


def _any16(v):
    pc = plsc.all_reduce_population_count(v)
    return lax.squeeze(lax.slice(pc, (0,), (1,)), (0,))


_mesh = plsc.VectorSubcoreMesh(core_axis_name="c", subcore_axis_name="s")


@functools.partial(
    pl.kernel,
    mesh=_mesh,
    compiler_params=pltpu.CompilerParams(needs_layout_passes=False),
    out_type=jax.ShapeDtypeStruct((N2 * F,), jnp.float32),
    scratch_types=[
        pltpu.VMEM((RB * F,), jnp.float32),
        pltpu.VMEM((16, 16, 128), jnp.float32),
        pltpu.VMEM((16, 128), jnp.int32),
        pltpu.VMEM((NB * 16,), jnp.int32),
        pltpu.VMEM((NB * 16,), jnp.int32),
        pltpu.VMEM((RB,), jnp.int32),
        pltpu.VMEM((16,), jnp.int32),
        pltpu.SemaphoreType.DMA,
        pltpu.SemaphoreType.DMA,
        pltpu.SemaphoreType.DMA,
    ],
)
def _segmax(m3, d3, blo, bhi, zflat, out, acc, mbuf, dbuf, blov, bhiv, tmp,
            remref, sem, sem2, sem3):
    c_ax = lax.axis_index("c")
    s_ax = lax.axis_index("s")
    wid = s_ax * 2 + c_ax
    iota = lax.iota(jnp.int32, 16)

    pltpu.sync_copy(blo, blov)
    pltpu.sync_copy(bhi, bhiv)

    for ib in range(2):
        b = wid * 2 + ib
        e_lo = blov[pl.ds(b * 16, 16)]
        e_hi = bhiv[pl.ds(b * 16, 16)]
        pltpu.async_copy(zflat, acc, sem3).wait()

        sbv = lax.shift_right_logical(e_lo, 11)
        row0d = sbv * 16
        row0m = sbv * 64

        def sb_body(carry, e_lo=e_lo, e_hi=e_hi, row0d=row0d, row0m=row0m):
            k, flag = carry
            pltpu.async_copy(d3.at[row0d + k * 16 + iota], dbuf, sem2).wait()
            base_e = (row0d + k * 16) * 128

            for c in range(4):
                pltpu.async_copy(m3.at[row0m + k * 64 + c * 16 + iota],
                                 mbuf, sem).wait()

                def g_body(g2, _, c=c, k=k, base_e=base_e, e_lo=e_lo,
                           e_hi=e_hi):
                    gg = c * 32 + g2
                    dv = dbuf[gg // 8, pl.ds((gg % 8) * 16, 16)]
                    e5 = g2 * 16 + iota
                    ev = base_e + c * 512 + e5
                    valid = (ev >= e_lo) & (ev < e_hi)
                    r_idx = lax.shift_right_logical(e5, 5)
                    s_idx = lax.shift_right_logical(e5 & 31, 1)
                    l0 = (e5 & 1) * 64
                    dvb = dv * 64
                    remref[...] = valid.astype(jnp.int32)

                    def wcond2(flag2):
                        return flag2 != 0

                    def wbody2(flag2):
                        rem = remref[...] != 0
                        plsc.store_scatter(tmp, [dv], iota, mask=rem)
                        t = plsc.load_gather(tmp, [dv], mask=rem)
                        win = rem & (t == iota)
                        for j0 in range(0, F, 8):
                            mvs = [plsc.load_gather(
                                mbuf, [r_idx, s_idx, l0 + j], mask=win)
                                for j in range(j0, j0 + 8)]
                            avs = [plsc.load_gather(acc, [dvb + j], mask=win)
                                   for j in range(j0, j0 + 8)]
                            for dj in range(8):
                                plsc.store_scatter(
                                    acc, [dvb + (j0 + dj)],
                                    jnp.maximum(avs[dj], mvs[dj]), mask=win)
                        nrem = rem & jnp.logical_not(win)
                        remref[...] = nrem.astype(jnp.int32)
                        return _any16(nrem)

                    lax.while_loop(wcond2, wbody2, _any16(valid))
                    return 0

                lax.fori_loop(0, 32, g_body, 0)

            nflag = _any16(base_e + 2048 < e_hi)
            return (k + 1, nflag)

        flag0 = _any16(row0d * 128 < e_hi)
        lax.while_loop(lambda cr: cr[1] != 0, sb_body, (jnp.int32(0), flag0))

        pltpu.sync_copy(acc, out.at[pl.ds(b * (RB * F), RB * F)])


_BW = EPAD // 32
_CH = 512


@functools.partial(
    pl.kernel,
    mesh=_mesh,
    compiler_params=pltpu.CompilerParams(needs_layout_passes=False),
    out_type=jax.ShapeDtypeStruct((EPAD, 128), jnp.float32),
    scratch_types=[
        pltpu.VMEM((_CH,), jnp.int32),
        pltpu.VMEM((_CH, 128), jnp.float32),
        pltpu.SemaphoreType.DMA,
    ],
)
def _gather128(table, idx, out, idxv, rows, sem):
    c_ax = lax.axis_index("c")
    s_ax = lax.axis_index("s")
    wid = s_ax * 2 + c_ax
    base = wid * _BW

    def body(k, _):
        off = base + k * _CH
        pltpu.sync_copy(idx.at[pl.ds(off, _CH)], idxv)
        pltpu.async_copy(table.at[idxv], rows, sem).wait()
        pltpu.sync_copy(rows, out.at[pl.ds(off, _CH)])
        return 0

    lax.fori_loop(0, _BW // _CH, body, 0)


def kernel(pos, edge_index, batch, c1_w1, c1_b1, c1_w2, c1_b2, c2_w1, c2_b1,
           c2_w2, c2_b2, r1_w, r1_b, r2_w, r2_b):
    mask = jnp.zeros((N,), dtype=bool).at[edge_index.reshape(-1)].set(True)
    assoc = jnp.cumsum(mask.astype(jnp.int32)) - 1
    assoc128 = jnp.broadcast_to(
        jnp.pad(assoc, (0, N2 - N))[:, None], (N2, 128))
    pad_i = (jnp.arange(EPAD - E, dtype=jnp.int32) % N2)
    src_raw = jnp.concatenate([edge_index[0], pad_i])
    dst_raw = jnp.concatenate([edge_index[1], pad_i])
    tblf = jax.lax.bitcast_convert_type(assoc128, jnp.float32)
    src = jax.lax.bitcast_convert_type(
        _gather128(tblf, src_raw), jnp.int32)[:E, 0]
    dst = jax.lax.bitcast_convert_type(
        _gather128(tblf, dst_raw), jnp.int32)[:E, 0]

    bucket = dst // RB
    skey = jnp.sort(bucket * 1048576 +
                    jnp.arange(E, dtype=jnp.int32))
    perm = skey & 0xFFFFF
    pad_idx = (jnp.arange(EPAD - E, dtype=jnp.int32) % N2)
    src_p = jnp.concatenate([src[perm], pad_idx])
    dst_p = jnp.concatenate([dst[perm], pad_idx])
    dstl_p = jnp.concatenate([(dst - bucket * RB)[perm],
                              jnp.zeros((EPAD - E,), jnp.int32)])
    sb = skey >> 20
    bstart = jnp.searchsorted(sb, jnp.arange(NB + 1, dtype=jnp.int32)
                              ).astype(jnp.int32)
    blo = jnp.repeat(bstart[:NB], 16)
    bhi = jnp.repeat(bstart[1:], 16)
    d3 = dstl_p.reshape(EPAD // 128, 128)
    zflat = jnp.zeros((RB * F,), jnp.float32)

    def segmax(m):
        hflat = _segmax(m.reshape(EPAD // 32, 16, 128), d3, blo, bhi, zflat)
        return hflat.reshape(N2, F)

    pos128 = jnp.zeros((N2, 128), jnp.float32).at[:N, :3].set(pos)
    gsrc = _gather128(pos128, src_p)
    gdst = _gather128(pos128, dst_p)
    w1a_1 = jnp.zeros((128, F), jnp.float32).at[:3].set(c1_w1[:3])
    w1b_1 = jnp.zeros((128, F), jnp.float32).at[:3].set(c1_w1[3:])
    wdb2 = jnp.zeros((128, F), jnp.float32).at[:3].set(c2_w1[F:])
    m1, db2 = _layer1_mlp(gsrc, gdst, w1a_1, w1b_1, c1_b1[None, :], c1_w2,
                          c1_b2[None, :], wdb2)
    h = segmax(m1)

    w1a_2 = jnp.zeros((128, F), jnp.float32).at[:F].set(c2_w1[:F])
    for _ in range(2):
        h128 = jnp.zeros((N2, 128), jnp.float32).at[:, :F].set(h)
        xsrc = _gather128(h128, src_p)
        m = _edge_mlp(xsrc, db2, w1a_2, c2_b1[None, :], c2_w2, c2_b2[None, :])
        h = segmax(m)

    hf = h[:N]
    sums = jax.ops.segment_sum(hf, batch, num_segments=G)
    cnt = jax.ops.segment_sum(jnp.ones((N,), jnp.float32), batch,
                              num_segments=G)
    mean = sums / jnp.maximum(cnt, 1.0)[:, None]
    return (mean @ r1_w + r1_b, mean @ r2_w + r2_b)
